# baseline (speedup 1.0000x reference)
import functools

import jax
import jax.numpy as jnp
from jax import lax
from jax.experimental import pallas as pl
from jax.experimental.pallas import tpu as pltpu
from jax.experimental.pallas import tpu_sc as plsc

_info = plsc.get_sparse_core_info()
_NC, _NS, _L = _info.num_cores, _info.num_subcores, _info.num_lanes
_NW = _NC * _NS


def _make(B, T, V, D):
    BW = B // _NW
    G = BW // _L
    P = 128 // D
    mesh = plsc.VectorSubcoreMesh(core_axis_name="c", subcore_axis_name="s")

    @functools.partial(
        pl.kernel,
        mesh=mesh,
        out_type=jax.ShapeDtypeStruct((T, D, B), jnp.float32),
        compiler_params=pltpu.CompilerParams(use_tc_tiling_on_sc=True, needs_layout_passes=False),
        scratch_types=[
            pltpu.VMEM((T, BW), jnp.int32),
            pltpu.VMEM((2, BW), jnp.int32),
            pltpu.VMEM((2, BW), jnp.int32),
            pltpu.VMEM((2, BW, 128), jnp.float32),
            pltpu.VMEM((2, D, BW), jnp.float32),
            pltpu.SemaphoreType.DMA,
            pltpu.SemaphoreType.DMA,
            pltpu.SemaphoreType.DMA,
            pltpu.SemaphoreType.DMA,
        ],
    )
    def k(ids_hbm, tq_hbm, out_hbm, ids_v, qidx_v, sub_v, gath_v, outb_v,
          gsem0, gsem1, osem0, osem1):
        gsems = (gsem0, gsem1)
        osems = (osem0, osem1)
        wid = lax.axis_index("s") * _NC + lax.axis_index("c")
        b0 = wid * BW
        pltpu.sync_copy(ids_hbm.at[:, pl.ds(b0, BW)], ids_v)
        lanes = lax.iota(jnp.int32, _L)

        def stage(t, buf):
            for g in range(G):
                ids16 = ids_v[t, pl.ds(g * _L, _L)]
                qidx_v[buf, pl.ds(g * _L, _L)] = lax.shift_right_logical(ids16, 2)
                sub_v[buf, pl.ds(g * _L, _L)] = lax.bitwise_and(ids16, P - 1)
            pltpu.async_copy(tq_hbm.at[qidx_v.at[buf]], gath_v.at[buf], gsems[buf])

        def extract(t, buf):
            for g in range(G):
                row16 = lanes + (g * _L)
                col0 = sub_v[buf, pl.ds(g * _L, _L)] * D
                for d in range(D):
                    v = plsc.load_gather(gath_v.at[buf], [row16, col0 + d])
                    outb_v[buf, d, pl.ds(g * _L, _L)] = v

        stage(0, 0)

        def body(t2, carry):
            for b in range(2):
                t = t2 * 2 + b
                if b == 0:
                    stage(t + 1, 1)
                else:
                    @pl.when(t2 < T // 2 - 1)
                    def _():
                        stage(t + 1, 0)
                pltpu.make_async_copy(
                    tq_hbm.at[qidx_v.at[b]], gath_v.at[b], gsems[b]
                ).wait()

                @pl.when(t2 >= 1)
                def _():
                    pltpu.make_async_copy(
                        outb_v.at[b], out_hbm.at[t, :, pl.ds(b0, BW)], osems[b]
                    ).wait()

                extract(t, b)
                pltpu.async_copy(
                    outb_v.at[b], out_hbm.at[t, :, pl.ds(b0, BW)], osems[b]
                )
            return carry

        lax.fori_loop(0, T // 2, body, 0)
        for b in range(2):
            pltpu.make_async_copy(
                outb_v.at[b], out_hbm.at[T - 2 + b, :, pl.ds(b0, BW)], osems[b]
            ).wait()

    return k


def kernel(ids_bt, emb_weight):
    B, T = ids_bt.shape
    V, D = emb_weight.shape
    ids_t = ids_bt.T.astype(jnp.int32)
    table_q = emb_weight.reshape(V * D // 128, 128)
    out_t = _make(B, T, V, D)(ids_t, table_q)
    return jnp.transpose(out_t, (2, 0, 1))


# speedup vs baseline: 1.1569x; 1.1569x over previous
import functools

import jax
import jax.numpy as jnp
from jax import lax
from jax.experimental import pallas as pl
from jax.experimental.pallas import tpu as pltpu
from jax.experimental.pallas import tpu_sc as plsc

_info = plsc.get_sparse_core_info()
_NC, _NS, _L = _info.num_cores, _info.num_subcores, _info.num_lanes
_NW = _NC * _NS


def _make(B, T, V, D):
    BW = B // _NW
    G = BW // _L
    P = 128 // D
    mesh = plsc.VectorSubcoreMesh(core_axis_name="c", subcore_axis_name="s")

    @functools.partial(
        pl.kernel,
        mesh=mesh,
        out_type=jax.ShapeDtypeStruct((T, D, B), jnp.float32),
        compiler_params=pltpu.CompilerParams(use_tc_tiling_on_sc=True, needs_layout_passes=False),
        scratch_types=[
            pltpu.VMEM((T, BW), jnp.int32),
            pltpu.VMEM((2, BW), jnp.int32),
            pltpu.VMEM((2, BW), jnp.int32),
            pltpu.VMEM((2, BW, 128), jnp.float32),
            pltpu.VMEM((2, D, BW), jnp.float32),
            pltpu.SemaphoreType.DMA,
            pltpu.SemaphoreType.DMA,
            pltpu.SemaphoreType.DMA,
            pltpu.SemaphoreType.DMA,
        ],
    )
    def k(ids_hbm, tq_hbm, out_hbm, ids_v, qidx_v, sub_v, gath_v, outb_v,
          gsem0, gsem1, osem0, osem1):
        gsems = (gsem0, gsem1)
        osems = (osem0, osem1)
        wid = lax.axis_index("s") * _NC + lax.axis_index("c")
        b0 = wid * BW
        pltpu.sync_copy(ids_hbm.at[:, pl.ds(b0, BW)], ids_v)
        lanes = lax.iota(jnp.int32, _L)

        def stage(t, buf):
            for g in range(G):
                ids16 = ids_v[t, pl.ds(g * _L, _L)]
                qidx_v[buf, pl.ds(g * _L, _L)] = lax.shift_right_logical(ids16, 2)
                sub_v[buf, pl.ds(g * _L, _L)] = lax.bitwise_and(ids16, P - 1)
            pltpu.async_copy(tq_hbm.at[qidx_v.at[buf]], gath_v.at[buf], gsems[buf])

        def extract(t, buf):
            rows = [lanes + (g * _L) for g in range(G)]
            col0s = [sub_v[buf, pl.ds(g * _L, _L)] * D for g in range(G)]
            for d in range(D):
                vs = [
                    plsc.load_gather(gath_v.at[buf], [rows[g], col0s[g] + d])
                    for g in range(G)
                ]
                for g in range(G):
                    outb_v[buf, d, pl.ds(g * _L, _L)] = vs[g]

        stage(0, 0)

        def body(t2, carry):
            for b in range(2):
                t = t2 * 2 + b
                if b == 0:
                    stage(t + 1, 1)
                else:
                    @pl.when(t2 < T // 2 - 1)
                    def _():
                        stage(t + 1, 0)
                pltpu.make_async_copy(
                    tq_hbm.at[qidx_v.at[b]], gath_v.at[b], gsems[b]
                ).wait()

                @pl.when(t2 >= 1)
                def _():
                    pltpu.make_async_copy(
                        outb_v.at[b], out_hbm.at[t, :, pl.ds(b0, BW)], osems[b]
                    ).wait()

                extract(t, b)
                pltpu.async_copy(
                    outb_v.at[b], out_hbm.at[t, :, pl.ds(b0, BW)], osems[b]
                )
            return carry

        lax.fori_loop(0, T // 2, body, 0)
        for b in range(2):
            pltpu.make_async_copy(
                outb_v.at[b], out_hbm.at[T - 2 + b, :, pl.ds(b0, BW)], osems[b]
            ).wait()

    return k


def kernel(ids_bt, emb_weight):
    B, T = ids_bt.shape
    V, D = emb_weight.shape
    ids_t = ids_bt.T.astype(jnp.int32)
    table_q = emb_weight.reshape(V * D // 128, 128)
    out_t = _make(B, T, V, D)(ids_t, table_q)
    return jnp.transpose(out_t, (2, 0, 1))


# speedup vs baseline: 1.1833x; 1.0229x over previous
import functools

import jax
import jax.numpy as jnp
from jax import lax
from jax.experimental import pallas as pl
from jax.experimental.pallas import tpu as pltpu
from jax.experimental.pallas import tpu_sc as plsc

_info = plsc.get_sparse_core_info()
_NC, _NS = _info.num_cores, _info.num_subcores
_NW = _NC * _NS


def _make_gather(V, D, N, chunk):
    n_per_w = N // _NW
    n_chunks = n_per_w // chunk
    mesh = plsc.VectorSubcoreMesh(core_axis_name="c", subcore_axis_name="s")

    @functools.partial(
        pl.kernel,
        mesh=mesh,
        out_type=jax.ShapeDtypeStruct((N, D), jnp.float32),
        compiler_params=pltpu.CompilerParams(use_tc_tiling_on_sc=False),
        scratch_types=[
            pltpu.VMEM((2, chunk), jnp.int32),
            pltpu.VMEM((2, chunk, D), jnp.float32),
            pltpu.SemaphoreType.DMA,
            pltpu.SemaphoreType.DMA,
            pltpu.SemaphoreType.DMA,
            pltpu.SemaphoreType.DMA,
        ],
    )
    def gather(ids_hbm, table_hbm, out_hbm, idx_v, rows_v, g0, g1, o0, o1):
        gsems = (g0, g1)
        osems = (o0, o1)
        wid = lax.axis_index("s") * _NC + lax.axis_index("c")
        base = wid * n_per_w

        def stage(i, b):
            off = base + i * chunk
            pltpu.sync_copy(ids_hbm.at[pl.ds(off, chunk)], idx_v.at[b])
            pltpu.async_copy(table_hbm.at[idx_v.at[b]], rows_v.at[b], gsems[b])

        stage(0, 0)

        def body(i2, carry):
            for b in range(2):
                i = i2 * 2 + b
                if b == 0:
                    stage(i + 1, 1)
                else:
                    @pl.when(i2 < n_chunks // 2 - 1)
                    def _():
                        stage(i + 1, 0)
                pltpu.make_async_copy(
                    table_hbm.at[idx_v.at[b]], rows_v.at[b], gsems[b]
                ).wait()
                off = base + i * chunk

                @pl.when(i2 >= 1)
                def _():
                    pltpu.make_async_copy(
                        rows_v.at[b], out_hbm.at[pl.ds(off, chunk)], osems[b]
                    ).wait()

                pltpu.async_copy(
                    rows_v.at[b], out_hbm.at[pl.ds(off, chunk)], osems[b]
                )
            return carry

        lax.fori_loop(0, n_chunks // 2, body, 0)
        for b in range(2):
            off = base + (n_chunks - 2 + b) * chunk
            pltpu.make_async_copy(
                rows_v.at[b], out_hbm.at[pl.ds(off, chunk)], osems[b]
            ).wait()

    return gather


def kernel(ids_bt, emb_weight):
    B, T = ids_bt.shape
    V, D = emb_weight.shape
    N = B * T
    ids_flat = ids_bt.reshape(N).astype(jnp.int32)
    table_q = lax.optimization_barrier(emb_weight.reshape(V * D // 128, 128))
    table = table_q.reshape(V, D)
    out = _make_gather(V, D, N, 1600)(ids_flat, table)
    return out.reshape(B, T, D)
